# Initial kernel scaffold; baseline (speedup 1.0000x reference)
import jax
import jax.numpy as jnp
from jax.experimental import pallas as pl


def kernel(x, x_node_inds, pe):
    raise NotImplementedError("write your pallas kernel here")


# speedup vs baseline: 1.4841x; 1.4841x over previous
import jax
import jax.numpy as jnp
from jax.experimental import pallas as pl

_BLK = 1024


def _body(idx_ref, x_ref, pe_ref, o_ref):
    idx = idx_ref[0, 0, :]
    onehot = (idx[:, None] == jax.lax.broadcasted_iota(jnp.int32, (_BLK, 96), 1)
              ).astype(jnp.float32)
    gathered = jnp.dot(onehot, pe_ref[...], preferred_element_type=jnp.float32)
    o_ref[...] = x_ref[...] + gathered


def kernel(x, x_node_inds, pe):
    B, S, D = x.shape
    N = B * S
    x2 = x.reshape(N, D)
    idx2 = jnp.tile(x_node_inds.astype(jnp.int32), B)
    n_blk = N // _BLK
    idx3 = idx2.reshape(n_blk, 1, _BLK)
    pe_pad = jnp.zeros((96, D), jnp.float32).at[:90].set(pe[0])

    out2 = pl.pallas_call(
        _body,
        grid=(n_blk,),
        in_specs=[
            pl.BlockSpec((1, 1, _BLK), lambda i: (i, 0, 0)),
            pl.BlockSpec((_BLK, D), lambda i: (i, 0)),
            pl.BlockSpec((96, D), lambda i: (0, 0)),
        ],
        out_specs=pl.BlockSpec((_BLK, D), lambda i: (i, 0)),
        out_shape=jax.ShapeDtypeStruct((N, D), jnp.float32),
    )(idx3, x2, pe_pad)
    return out2.reshape(B, S, D)
